# baseline (speedup 1.0000x reference)
import functools

import jax
import jax.numpy as jnp
from jax import lax
from jax.experimental import pallas as pl
from jax.experimental.pallas import tpu as pltpu
from jax.experimental.pallas import tpu_sc as plsc

NC = 2
NS = 16
NW = NC * NS
LANES = 16

D = 8
AUG = 16
AUG_W = 8
SUB = 128
NSUB = 8
CHUNK = SUB * NSUB

S_BLK = 256


def _ceil_to(x, m):
    return (x + m - 1) // m * m


def _vsqrt(x):
    x = jnp.maximum(x, 1e-24)
    i = plsc.bitcast(x, jnp.int32)
    i = jnp.int32(0x5F3759DF) - lax.shift_right_logical(i, 1)
    r = plsc.bitcast(i, jnp.float32)
    r = r * (1.5 - 0.5 * x * r * r)
    r = r * (1.5 - 0.5 * x * r * r)
    r = r * (1.5 - 0.5 * x * r * r)
    return x * r


def _wid():
    return lax.axis_index("s") * NC + lax.axis_index("c")


def _link_kernel_body(e_per_tile, aug_i, aug_j, idx_i, idx_j, out,
                      idx_iv, idx_jv, rows_iv, rows_jv, acc_v, sem_i, sem_j):
    wid = _wid()
    base_e = wid * e_per_tile
    iota = lax.iota(jnp.int32, LANES)
    n_full = e_per_tile // CHUNK
    tail = e_per_tile - n_full * CHUNK
    tail_subs, off = [], 0
    while off < tail:
        sz = min(SUB, tail - off)
        tail_subs.append((off, sz))
        off += sz
    full_subs = [(k * SUB, SUB) for k in range(NSUB)]

    def fire(eb, count, subs):
        pltpu.sync_copy(idx_i.at[pl.ds(eb, count)],
                        idx_iv.at[pl.ds(0, count)])
        pltpu.sync_copy(idx_j.at[pl.ds(eb, count)],
                        idx_jv.at[pl.ds(0, count)])
        cps = []
        for o, sz in subs:
            cps.append(pltpu.async_copy(
                aug_i.at[idx_iv.at[pl.ds(o, sz)]],
                rows_iv.at[pl.ds(o, sz)], sem_i))
            cps.append(pltpu.async_copy(
                aug_j.at[idx_jv.at[pl.ds(o, sz)]],
                rows_jv.at[pl.ds(o, sz)], sem_j))
        return cps

    def group_body(g, acc):
        r = g * LANES + iota
        sq = []
        for w in range(D // 2):
            col = jnp.full((LANES,), w, jnp.int32)
            aw = plsc.load_gather(rows_iv, [r, col])
            bw = plsc.load_gather(rows_jv, [r, col])
            a0, a1 = plsc.unpack(plsc.bitcast(aw, jnp.bfloat16),
                                 format=plsc.PackFormat.INTERLEAVED)
            b0, b1 = plsc.unpack(plsc.bitcast(bw, jnp.bfloat16),
                                 format=plsc.PackFormat.INTERLEAVED)
            diff0 = a0 - b0 + 1e-6
            diff1 = a1 - b1 + 1e-6
            sq.append(diff0 * diff0)
            sq.append(diff1 * diff1)
        d2 = (((sq[0] + sq[1]) + (sq[2] + sq[3]))
              + ((sq[4] + sq[5]) + (sq[6] + sq[7])))
        colb = jnp.full((LANES,), D // 2, jnp.int32)
        aw4 = plsc.load_gather(rows_iv, [r, colb])
        bw4 = plsc.load_gather(rows_jv, [r, colb])
        beta_v, _ = plsc.unpack(plsc.bitcast(aw4, jnp.bfloat16),
                                format=plsc.PackFormat.INTERLEAVED)
        gamma_v, _ = plsc.unpack(plsc.bitcast(bw4, jnp.bfloat16),
                                 format=plsc.PackFormat.INTERLEAVED)
        return acc + (beta_v + gamma_v - _vsqrt(d2))

    def chunk_body(c, acc):
        for cp in fire(base_e + c * CHUNK, CHUNK, full_subs):
            cp.wait()
        return lax.fori_loop(0, CHUNK // LANES, group_body, acc)

    acc = lax.fori_loop(0, n_full, chunk_body,
                        jnp.zeros((LANES,), jnp.float32))
    if tail:
        for cp in fire(base_e + n_full * CHUNK, tail, tail_subs):
            cp.wait()
        acc = lax.fori_loop(0, tail // LANES, group_body, acc)
    acc_v[...] = acc
    pltpu.sync_copy(acc_v, out.at[wid])


def _sample_kernel_body(s_per_tile, aug_i, aug_j, idx_i, idx_j, out_i, out_j,
                        idx_v, rows_v, sem):
    wid = _wid()
    base = wid * s_per_tile
    pltpu.sync_copy(idx_i.at[pl.ds(base, s_per_tile)], idx_v)
    pltpu.async_copy(aug_i.at[idx_v], rows_v, sem).wait()
    pltpu.sync_copy(rows_v, out_i.at[pl.ds(base, s_per_tile)])
    pltpu.sync_copy(idx_j.at[pl.ds(base, s_per_tile)], idx_v)
    pltpu.async_copy(aug_j.at[idx_v], rows_v, sem).wait()
    pltpu.sync_copy(rows_v, out_j.at[pl.ds(base, s_per_tile)])


def _dense_body(s_i, s_j, a_ref, b_ref, o_ref):
    i = pl.program_id(0)
    a = a_ref[...]
    b = b_ref[...]
    az = a[:, :D] + 1e-6
    bz = b[:, :D]
    a2 = jnp.sum(az * az, axis=1, keepdims=True)
    b2 = jnp.sum(bz * bz, axis=1)[None, :]
    cross = lax.dot_general(az, bz, (((1,), (1,)), ((), ())),
                            preferred_element_type=jnp.float32)
    d2 = jnp.maximum(a2 + b2 - 2.0 * cross, 0.0)
    lam = a[:, D][:, None] + b[:, D][None, :] - jnp.sqrt(d2)
    n_pad = b.shape[0]
    rows = i * S_BLK + lax.broadcasted_iota(jnp.int32, (S_BLK, n_pad), 0)
    cols = lax.broadcasted_iota(jnp.int32, (S_BLK, n_pad), 1)
    val = jnp.sum(jnp.where((rows < s_i) & (cols < s_j), jnp.exp(lam), 0.0))

    @pl.when(i == 0)
    def _():
        o_ref[...] = jnp.zeros((1, 1), jnp.float32)

    o_ref[...] = o_ref[...] + val


def kernel(beta, gamma, latent_zi, latent_zj, sample_i_idx, sample_j_idx,
           sparse_i_sample, sparse_j_sample):
    n_i, d = latent_zi.shape
    n_j, _ = latent_zj.shape
    s_i = sample_i_idx.shape[0]
    s_j = sample_j_idx.shape[0]
    e = sparse_i_sample.shape[0]
    f32 = jnp.float32

    bf = jnp.bfloat16

    def _aug(tab, bias):
        a = jnp.concatenate(
            [tab.astype(bf), bias[:, None].astype(bf),
             jnp.zeros((tab.shape[0], AUG - d - 1), bf)], axis=1)
        return lax.bitcast_convert_type(
            a.reshape(tab.shape[0], AUG_W, 2), jnp.int32)

    aug_i = _aug(latent_zi, beta)
    aug_j = _aug(latent_zj, gamma)

    e_per_tile = e // NW

    sc_params = pltpu.CompilerParams(use_tc_tiling_on_sc=False,
                                     needs_layout_passes=False)
    mesh = plsc.VectorSubcoreMesh(core_axis_name="c", subcore_axis_name="s",
                                  num_cores=NC, num_subcores=NS)
    link_fn = pl.kernel(
        functools.partial(_link_kernel_body, e_per_tile),
        out_type=jax.ShapeDtypeStruct((NW, LANES), f32),
        mesh=mesh,
        compiler_params=sc_params,
        scratch_types=[
            pltpu.VMEM((CHUNK,), jnp.int32),
            pltpu.VMEM((CHUNK,), jnp.int32),
            pltpu.VMEM((CHUNK, AUG_W), jnp.int32),
            pltpu.VMEM((CHUNK, AUG_W), jnp.int32),
            pltpu.VMEM((LANES,), f32),
            pltpu.SemaphoreType.DMA,
            pltpu.SemaphoreType.DMA,
        ],
    )
    s_pad = _ceil_to(max(s_i, s_j), NW * 8)
    s_per_tile = s_pad // NW
    sidx_i = jnp.pad(sample_i_idx.astype(jnp.int32), (0, s_pad - s_i))
    sidx_j = jnp.pad(sample_j_idx.astype(jnp.int32), (0, s_pad - s_j))
    sample_fn = pl.kernel(
        functools.partial(_sample_kernel_body, s_per_tile),
        out_type=(jax.ShapeDtypeStruct((s_pad, AUG_W), jnp.int32),
                  jax.ShapeDtypeStruct((s_pad, AUG_W), jnp.int32)),
        mesh=plsc.VectorSubcoreMesh(core_axis_name="c", subcore_axis_name="s",
                                    num_cores=NC, num_subcores=NS),
        compiler_params=sc_params,
        scratch_types=[
            pltpu.VMEM((s_per_tile,), jnp.int32),
            pltpu.VMEM((s_per_tile, AUG_W), jnp.int32),
            pltpu.SemaphoreType.DMA,
        ],
    )
    rows_i_w, rows_j_w = sample_fn(aug_i, aug_j, sidx_i, sidx_j)
    rows_i_s = lax.bitcast_convert_type(rows_i_w, bf).reshape(
        s_pad, AUG).astype(f32)
    rows_j_s = lax.bitcast_convert_type(rows_j_w, bf).reshape(
        s_pad, AUG).astype(f32)

    link_partials = link_fn(aug_i, aug_j,
                            sparse_i_sample.astype(jnp.int32),
                            sparse_j_sample.astype(jnp.int32))

    exp_sum = pl.pallas_call(
        functools.partial(_dense_body, s_i, s_j),
        grid=(s_pad // S_BLK,),
        in_specs=[
            pl.BlockSpec((S_BLK, AUG), lambda i: (i, 0)),
            pl.BlockSpec((s_pad, AUG), lambda i: (0, 0)),
        ],
        out_specs=pl.BlockSpec((1, 1), lambda i: (0, 0)),
        out_shape=jax.ShapeDtypeStruct((1, 1), f32),
    )(rows_i_s, rows_j_s)

    return jnp.sum(link_partials) - exp_sum[0, 0]


# speedup vs baseline: 1.3477x; 1.3477x over previous
import functools

import jax
import jax.numpy as jnp
from jax import lax
from jax.experimental import pallas as pl
from jax.experimental.pallas import tpu as pltpu
from jax.experimental.pallas import tpu_sc as plsc

NC = 2
NS = 16
NW = NC * NS
LANES = 16

D = 8
AUG = 16
SUB = 128
NSUB = 16
CHUNK = SUB * NSUB

S_BLK = 256


def _ceil_to(x, m):
    return (x + m - 1) // m * m


def _vsqrt(x):
    x = jnp.maximum(x, 1e-24)
    i = plsc.bitcast(x, jnp.int32)
    i = jnp.int32(0x5F3759DF) - lax.shift_right_logical(i, 1)
    r = plsc.bitcast(i, jnp.float32)
    r = r * (1.5 - 0.5 * x * r * r)
    r = r * (1.5 - 0.5 * x * r * r)
    r = r * (1.5 - 0.5 * x * r * r)
    return x * r


def _wid():
    return lax.axis_index("s") * NC + lax.axis_index("c")


def _link_kernel_body(e_per_tile, aug_i, aug_j, idx_i, idx_j, out,
                      idx_iv, idx_jv, rows_iv, rows_jv, acc_v, sem_i, sem_j):
    wid = _wid()
    base_e = wid * e_per_tile
    iota = lax.iota(jnp.int32, LANES)
    n_full = e_per_tile // CHUNK
    tail = e_per_tile - n_full * CHUNK
    tail_subs, off = [], 0
    while off < tail:
        sz = min(SUB, tail - off)
        tail_subs.append((off, sz))
        off += sz
    full_subs = [(k * SUB, SUB) for k in range(NSUB)]

    def fire(eb, count, subs):
        pltpu.sync_copy(idx_i.at[pl.ds(eb, count)],
                        idx_iv.at[pl.ds(0, count)])
        pltpu.sync_copy(idx_j.at[pl.ds(eb, count)],
                        idx_jv.at[pl.ds(0, count)])
        cps = []
        for o, sz in subs:
            cps.append(pltpu.async_copy(
                aug_i.at[idx_iv.at[pl.ds(o, sz)]],
                rows_iv.at[pl.ds(o, sz)], sem_i))
            cps.append(pltpu.async_copy(
                aug_j.at[idx_jv.at[pl.ds(o, sz)]],
                rows_jv.at[pl.ds(o, sz)], sem_j))
        return cps

    def group_body(g, acc):
        r = g * LANES + iota
        sq = []
        for d in range(D):
            col = jnp.full((LANES,), d, jnp.int32)
            a = plsc.load_gather(rows_iv, [r, col])
            b = plsc.load_gather(rows_jv, [r, col])
            diff = a - b + 1e-6
            sq.append(diff * diff)
        d2 = (((sq[0] + sq[1]) + (sq[2] + sq[3]))
              + ((sq[4] + sq[5]) + (sq[6] + sq[7])))
        colb = jnp.full((LANES,), D, jnp.int32)
        beta_v = plsc.load_gather(rows_iv, [r, colb])
        gamma_v = plsc.load_gather(rows_jv, [r, colb])
        return acc + (beta_v + gamma_v - _vsqrt(d2))

    def chunk_body(c, acc):
        for cp in fire(base_e + c * CHUNK, CHUNK, full_subs):
            cp.wait()
        return lax.fori_loop(0, CHUNK // LANES, group_body, acc)

    acc = lax.fori_loop(0, n_full, chunk_body,
                        jnp.zeros((LANES,), jnp.float32))
    if tail:
        for cp in fire(base_e + n_full * CHUNK, tail, tail_subs):
            cp.wait()
        acc = lax.fori_loop(0, tail // LANES, group_body, acc)
    acc_v[...] = acc
    pltpu.sync_copy(acc_v, out.at[wid])


def _sample_kernel_body(s_per_tile, aug_i, aug_j, idx_i, idx_j, out_i, out_j,
                        idx_v, rows_v, sem):
    wid = _wid()
    base = wid * s_per_tile
    pltpu.sync_copy(idx_i.at[pl.ds(base, s_per_tile)], idx_v)
    pltpu.async_copy(aug_i.at[idx_v], rows_v, sem).wait()
    pltpu.sync_copy(rows_v, out_i.at[pl.ds(base, s_per_tile)])
    pltpu.sync_copy(idx_j.at[pl.ds(base, s_per_tile)], idx_v)
    pltpu.async_copy(aug_j.at[idx_v], rows_v, sem).wait()
    pltpu.sync_copy(rows_v, out_j.at[pl.ds(base, s_per_tile)])


def _dense_body(s_i, s_j, a_ref, b_ref, o_ref):
    i = pl.program_id(0)
    a = a_ref[...]
    b = b_ref[...]
    az = a[:, :D] + 1e-6
    bz = b[:, :D]
    a2 = jnp.sum(az * az, axis=1, keepdims=True)
    b2 = jnp.sum(bz * bz, axis=1)[None, :]
    cross = lax.dot_general(az, bz, (((1,), (1,)), ((), ())),
                            preferred_element_type=jnp.float32)
    d2 = jnp.maximum(a2 + b2 - 2.0 * cross, 0.0)
    lam = a[:, D][:, None] + b[:, D][None, :] - jnp.sqrt(d2)
    n_pad = b.shape[0]
    rows = i * S_BLK + lax.broadcasted_iota(jnp.int32, (S_BLK, n_pad), 0)
    cols = lax.broadcasted_iota(jnp.int32, (S_BLK, n_pad), 1)
    val = jnp.sum(jnp.where((rows < s_i) & (cols < s_j), jnp.exp(lam), 0.0))

    @pl.when(i == 0)
    def _():
        o_ref[...] = jnp.zeros((1, 1), jnp.float32)

    o_ref[...] = o_ref[...] + val


def kernel(beta, gamma, latent_zi, latent_zj, sample_i_idx, sample_j_idx,
           sparse_i_sample, sparse_j_sample):
    n_i, d = latent_zi.shape
    n_j, _ = latent_zj.shape
    s_i = sample_i_idx.shape[0]
    s_j = sample_j_idx.shape[0]
    e = sparse_i_sample.shape[0]
    f32 = jnp.float32

    aug_i = jnp.concatenate(
        [latent_zi, beta[:, None], jnp.zeros((n_i, AUG - d - 1), f32)], axis=1)
    aug_j = jnp.concatenate(
        [latent_zj, gamma[:, None], jnp.zeros((n_j, AUG - d - 1), f32)], axis=1)

    e_per_tile = e // NW

    sc_params = pltpu.CompilerParams(use_tc_tiling_on_sc=False,
                                     needs_layout_passes=False)
    mesh = plsc.VectorSubcoreMesh(core_axis_name="c", subcore_axis_name="s",
                                  num_cores=NC, num_subcores=NS)
    link_fn = pl.kernel(
        functools.partial(_link_kernel_body, e_per_tile),
        out_type=jax.ShapeDtypeStruct((NW, LANES), f32),
        mesh=mesh,
        compiler_params=sc_params,
        scratch_types=[
            pltpu.VMEM((CHUNK,), jnp.int32),
            pltpu.VMEM((CHUNK,), jnp.int32),
            pltpu.VMEM((CHUNK, AUG), f32),
            pltpu.VMEM((CHUNK, AUG), f32),
            pltpu.VMEM((LANES,), f32),
            pltpu.SemaphoreType.DMA,
            pltpu.SemaphoreType.DMA,
        ],
    )
    s_pad = _ceil_to(max(s_i, s_j), NW * 8)
    s_per_tile = s_pad // NW
    sidx_i = jnp.pad(sample_i_idx.astype(jnp.int32), (0, s_pad - s_i))
    sidx_j = jnp.pad(sample_j_idx.astype(jnp.int32), (0, s_pad - s_j))
    sample_fn = pl.kernel(
        functools.partial(_sample_kernel_body, s_per_tile),
        out_type=(jax.ShapeDtypeStruct((s_pad, AUG), f32),
                  jax.ShapeDtypeStruct((s_pad, AUG), f32)),
        mesh=plsc.VectorSubcoreMesh(core_axis_name="c", subcore_axis_name="s",
                                    num_cores=NC, num_subcores=NS),
        compiler_params=sc_params,
        scratch_types=[
            pltpu.VMEM((s_per_tile,), jnp.int32),
            pltpu.VMEM((s_per_tile, AUG), f32),
            pltpu.SemaphoreType.DMA,
        ],
    )
    rows_i_s, rows_j_s = sample_fn(aug_i, aug_j, sidx_i, sidx_j)

    link_partials = link_fn(aug_i, aug_j,
                            sparse_i_sample.astype(jnp.int32),
                            sparse_j_sample.astype(jnp.int32))

    exp_sum = pl.pallas_call(
        functools.partial(_dense_body, s_i, s_j),
        grid=(s_pad // S_BLK,),
        in_specs=[
            pl.BlockSpec((S_BLK, AUG), lambda i: (i, 0)),
            pl.BlockSpec((s_pad, AUG), lambda i: (0, 0)),
        ],
        out_specs=pl.BlockSpec((1, 1), lambda i: (0, 0)),
        out_shape=jax.ShapeDtypeStruct((1, 1), f32),
    )(rows_i_s, rows_j_s)

    return jnp.sum(link_partials) - exp_sum[0, 0]


# speedup vs baseline: 1.3729x; 1.0187x over previous
import functools

import jax
import jax.numpy as jnp
from jax import lax
from jax.experimental import pallas as pl
from jax.experimental.pallas import tpu as pltpu
from jax.experimental.pallas import tpu_sc as plsc

NC = 2
NS = 16
NW = NC * NS
LANES = 16

D = 8
AUG = 16
SUB = 128
NSUB = 24
CHUNK = SUB * NSUB

S_BLK = 256


def _ceil_to(x, m):
    return (x + m - 1) // m * m


def _vsqrt(x):
    x = jnp.maximum(x, 1e-24)
    i = plsc.bitcast(x, jnp.int32)
    i = jnp.int32(0x5F3759DF) - lax.shift_right_logical(i, 1)
    r = plsc.bitcast(i, jnp.float32)
    r = r * (1.5 - 0.5 * x * r * r)
    r = r * (1.5 - 0.5 * x * r * r)
    r = r * (1.5 - 0.5 * x * r * r)
    return x * r


def _wid():
    return lax.axis_index("s") * NC + lax.axis_index("c")


def _link_kernel_body(e_per_tile, aug_i, aug_j, idx_i, idx_j, out,
                      idx_iv, idx_jv, rows_iv, rows_jv, acc_v, sem_i, sem_j):
    wid = _wid()
    base_e = wid * e_per_tile
    iota = lax.iota(jnp.int32, LANES)
    n_full = e_per_tile // CHUNK
    tail = e_per_tile - n_full * CHUNK
    tail_subs, off = [], 0
    while off < tail:
        sz = min(SUB, tail - off)
        tail_subs.append((off, sz))
        off += sz
    full_subs = [(k * SUB, SUB) for k in range(NSUB)]

    def fire(eb, count, subs):
        pltpu.sync_copy(idx_i.at[pl.ds(eb, count)],
                        idx_iv.at[pl.ds(0, count)])
        pltpu.sync_copy(idx_j.at[pl.ds(eb, count)],
                        idx_jv.at[pl.ds(0, count)])
        cps = []
        for o, sz in subs:
            cps.append(pltpu.async_copy(
                aug_i.at[idx_iv.at[pl.ds(o, sz)]],
                rows_iv.at[pl.ds(o, sz)], sem_i))
            cps.append(pltpu.async_copy(
                aug_j.at[idx_jv.at[pl.ds(o, sz)]],
                rows_jv.at[pl.ds(o, sz)], sem_j))
        return cps

    def group_body(g, acc):
        r = g * LANES + iota
        sq = []
        for d in range(D):
            col = jnp.full((LANES,), d, jnp.int32)
            a = plsc.load_gather(rows_iv, [r, col])
            b = plsc.load_gather(rows_jv, [r, col])
            diff = a - b + 1e-6
            sq.append(diff * diff)
        d2 = (((sq[0] + sq[1]) + (sq[2] + sq[3]))
              + ((sq[4] + sq[5]) + (sq[6] + sq[7])))
        colb = jnp.full((LANES,), D, jnp.int32)
        beta_v = plsc.load_gather(rows_iv, [r, colb])
        gamma_v = plsc.load_gather(rows_jv, [r, colb])
        return acc + (beta_v + gamma_v - _vsqrt(d2))

    def chunk_body(c, acc):
        for cp in fire(base_e + c * CHUNK, CHUNK, full_subs):
            cp.wait()
        return lax.fori_loop(0, CHUNK // LANES, group_body, acc)

    acc = lax.fori_loop(0, n_full, chunk_body,
                        jnp.zeros((LANES,), jnp.float32))
    if tail:
        for cp in fire(base_e + n_full * CHUNK, tail, tail_subs):
            cp.wait()
        acc = lax.fori_loop(0, tail // LANES, group_body, acc)
    acc_v[...] = acc
    pltpu.sync_copy(acc_v, out.at[wid])


def _sample_kernel_body(s_per_tile, aug_i, aug_j, idx_i, idx_j, out_i, out_j,
                        idx_v, rows_v, sem):
    wid = _wid()
    base = wid * s_per_tile
    pltpu.sync_copy(idx_i.at[pl.ds(base, s_per_tile)], idx_v)
    pltpu.async_copy(aug_i.at[idx_v], rows_v, sem).wait()
    pltpu.sync_copy(rows_v, out_i.at[pl.ds(base, s_per_tile)])
    pltpu.sync_copy(idx_j.at[pl.ds(base, s_per_tile)], idx_v)
    pltpu.async_copy(aug_j.at[idx_v], rows_v, sem).wait()
    pltpu.sync_copy(rows_v, out_j.at[pl.ds(base, s_per_tile)])


def _dense_body(s_i, s_j, a_ref, b_ref, o_ref):
    i = pl.program_id(0)
    a = a_ref[...]
    b = b_ref[...]
    az = a[:, :D] + 1e-6
    bz = b[:, :D]
    a2 = jnp.sum(az * az, axis=1, keepdims=True)
    b2 = jnp.sum(bz * bz, axis=1)[None, :]
    cross = lax.dot_general(az, bz, (((1,), (1,)), ((), ())),
                            preferred_element_type=jnp.float32)
    d2 = jnp.maximum(a2 + b2 - 2.0 * cross, 0.0)
    lam = a[:, D][:, None] + b[:, D][None, :] - jnp.sqrt(d2)
    n_pad = b.shape[0]
    rows = i * S_BLK + lax.broadcasted_iota(jnp.int32, (S_BLK, n_pad), 0)
    cols = lax.broadcasted_iota(jnp.int32, (S_BLK, n_pad), 1)
    val = jnp.sum(jnp.where((rows < s_i) & (cols < s_j), jnp.exp(lam), 0.0))

    @pl.when(i == 0)
    def _():
        o_ref[...] = jnp.zeros((1, 1), jnp.float32)

    o_ref[...] = o_ref[...] + val


def kernel(beta, gamma, latent_zi, latent_zj, sample_i_idx, sample_j_idx,
           sparse_i_sample, sparse_j_sample):
    n_i, d = latent_zi.shape
    n_j, _ = latent_zj.shape
    s_i = sample_i_idx.shape[0]
    s_j = sample_j_idx.shape[0]
    e = sparse_i_sample.shape[0]
    f32 = jnp.float32

    aug_i = jnp.concatenate(
        [latent_zi, beta[:, None], jnp.zeros((n_i, AUG - d - 1), f32)], axis=1)
    aug_j = jnp.concatenate(
        [latent_zj, gamma[:, None], jnp.zeros((n_j, AUG - d - 1), f32)], axis=1)

    e_per_tile = e // NW

    sc_params = pltpu.CompilerParams(use_tc_tiling_on_sc=False,
                                     needs_layout_passes=False)
    mesh = plsc.VectorSubcoreMesh(core_axis_name="c", subcore_axis_name="s",
                                  num_cores=NC, num_subcores=NS)
    link_fn = pl.kernel(
        functools.partial(_link_kernel_body, e_per_tile),
        out_type=jax.ShapeDtypeStruct((NW, LANES), f32),
        mesh=mesh,
        compiler_params=sc_params,
        scratch_types=[
            pltpu.VMEM((CHUNK,), jnp.int32),
            pltpu.VMEM((CHUNK,), jnp.int32),
            pltpu.VMEM((CHUNK, AUG), f32),
            pltpu.VMEM((CHUNK, AUG), f32),
            pltpu.VMEM((LANES,), f32),
            pltpu.SemaphoreType.DMA,
            pltpu.SemaphoreType.DMA,
        ],
    )
    s_pad = _ceil_to(max(s_i, s_j), NW * 8)
    s_per_tile = s_pad // NW
    sidx_i = jnp.pad(sample_i_idx.astype(jnp.int32), (0, s_pad - s_i))
    sidx_j = jnp.pad(sample_j_idx.astype(jnp.int32), (0, s_pad - s_j))
    sample_fn = pl.kernel(
        functools.partial(_sample_kernel_body, s_per_tile),
        out_type=(jax.ShapeDtypeStruct((s_pad, AUG), f32),
                  jax.ShapeDtypeStruct((s_pad, AUG), f32)),
        mesh=plsc.VectorSubcoreMesh(core_axis_name="c", subcore_axis_name="s",
                                    num_cores=NC, num_subcores=NS),
        compiler_params=sc_params,
        scratch_types=[
            pltpu.VMEM((s_per_tile,), jnp.int32),
            pltpu.VMEM((s_per_tile, AUG), f32),
            pltpu.SemaphoreType.DMA,
        ],
    )
    rows_i_s, rows_j_s = sample_fn(aug_i, aug_j, sidx_i, sidx_j)

    link_partials = link_fn(aug_i, aug_j,
                            sparse_i_sample.astype(jnp.int32),
                            sparse_j_sample.astype(jnp.int32))

    exp_sum = pl.pallas_call(
        functools.partial(_dense_body, s_i, s_j),
        grid=(s_pad // S_BLK,),
        in_specs=[
            pl.BlockSpec((S_BLK, AUG), lambda i: (i, 0)),
            pl.BlockSpec((s_pad, AUG), lambda i: (0, 0)),
        ],
        out_specs=pl.BlockSpec((1, 1), lambda i: (0, 0)),
        out_shape=jax.ShapeDtypeStruct((1, 1), f32),
    )(rows_i_s, rows_j_s)

    return jnp.sum(link_partials) - exp_sum[0, 0]


# speedup vs baseline: 1.6499x; 1.2017x over previous
import functools

import jax
import jax.numpy as jnp
from jax import lax
from jax.experimental import pallas as pl
from jax.experimental.pallas import tpu as pltpu
from jax.experimental.pallas import tpu_sc as plsc

NC = 2
NS = 16
NW = NC * NS
LANES = 16

D = 8
AUG = 16
SUB = 128
NSUB = 12
CHUNK = SUB * NSUB

S_BLK = 256


def _ceil_to(x, m):
    return (x + m - 1) // m * m


def _vsqrt(x):
    x = jnp.maximum(x, 1e-24)
    i = plsc.bitcast(x, jnp.int32)
    i = jnp.int32(0x5F3759DF) - lax.shift_right_logical(i, 1)
    r = plsc.bitcast(i, jnp.float32)
    r = r * (1.5 - 0.5 * x * r * r)
    r = r * (1.5 - 0.5 * x * r * r)
    r = r * (1.5 - 0.5 * x * r * r)
    return x * r


def _wid():
    return lax.axis_index("s") * NC + lax.axis_index("c")


def _link_kernel_body(e_per_tile, aug_i, aug_j, idx_i, idx_j, out,
                      idx_iv0, idx_jv0, rows_iv0, rows_jv0,
                      idx_iv1, idx_jv1, rows_iv1, rows_jv1,
                      acc_v, sem_i0, sem_j0, sem_i1, sem_j1):
    wid = _wid()
    base_e = wid * e_per_tile
    iota = lax.iota(jnp.int32, LANES)
    n_full = e_per_tile // CHUNK
    tail = e_per_tile - n_full * CHUNK
    tail_subs, off = [], 0
    while off < tail:
        sz = min(SUB, tail - off)
        tail_subs.append((off, sz))
        off += sz
    full_subs = [(k * SUB, SUB) for k in range(NSUB)]
    bufs = ((idx_iv0, idx_jv0, rows_iv0, rows_jv0, sem_i0, sem_j0),
            (idx_iv1, idx_jv1, rows_iv1, rows_jv1, sem_i1, sem_j1))

    def fire(eb, count, subs, buf):
        idx_iv, idx_jv, rows_iv, rows_jv, sem_i, sem_j = bufs[buf]
        pltpu.sync_copy(idx_i.at[pl.ds(eb, count)],
                        idx_iv.at[pl.ds(0, count)])
        pltpu.sync_copy(idx_j.at[pl.ds(eb, count)],
                        idx_jv.at[pl.ds(0, count)])
        cps = []
        for o, sz in subs:
            cps.append(pltpu.async_copy(
                aug_i.at[idx_iv.at[pl.ds(o, sz)]],
                rows_iv.at[pl.ds(o, sz)], sem_i))
            cps.append(pltpu.async_copy(
                aug_j.at[idx_jv.at[pl.ds(o, sz)]],
                rows_jv.at[pl.ds(o, sz)], sem_j))
        return cps

    def compute(n_groups, buf, acc):
        rows_iv, rows_jv = bufs[buf][2], bufs[buf][3]

        def group_body(g, acc):
            r = g * LANES + iota
            sq = []
            for d in range(D):
                col = jnp.full((LANES,), d, jnp.int32)
                a = plsc.load_gather(rows_iv, [r, col])
                b = plsc.load_gather(rows_jv, [r, col])
                diff = a - b + 1e-6
                sq.append(diff * diff)
            d2 = (((sq[0] + sq[1]) + (sq[2] + sq[3]))
                  + ((sq[4] + sq[5]) + (sq[6] + sq[7])))
            colb = jnp.full((LANES,), D, jnp.int32)
            beta_v = plsc.load_gather(rows_iv, [r, colb])
            gamma_v = plsc.load_gather(rows_jv, [r, colb])
            return acc + (beta_v + gamma_v - _vsqrt(d2))

        return lax.fori_loop(0, n_groups, group_body, acc)

    def drain(buf):
        idx_iv, idx_jv, rows_iv, rows_jv, sem_i, sem_j = bufs[buf]
        for o, sz in full_subs:
            pltpu.make_async_copy(
                aug_i.at[idx_iv.at[pl.ds(o, sz)]],
                rows_iv.at[pl.ds(o, sz)], sem_i).wait()
            pltpu.make_async_copy(
                aug_j.at[idx_jv.at[pl.ds(o, sz)]],
                rows_jv.at[pl.ds(o, sz)], sem_j).wait()

    n_pairs = n_full // 2
    fire(base_e, CHUNK, full_subs, 0)

    def outer(t, acc):
        for b in (0, 1):
            c = 2 * t + b

            @pl.when(c + 1 < n_full)
            def _():
                fire(base_e + (c + 1) * CHUNK, CHUNK, full_subs, 1 - b)

            drain(b)
            acc = compute(CHUNK // LANES, b, acc)
        return acc

    acc = lax.fori_loop(0, n_pairs, outer, jnp.zeros((LANES,), jnp.float32))
    if tail:
        for cp in fire(base_e + n_full * CHUNK, tail, tail_subs, 0):
            cp.wait()
        acc = compute(tail // LANES, 0, acc)
    acc_v[...] = acc
    pltpu.sync_copy(acc_v, out.at[wid])


def _sample_kernel_body(s_per_tile, aug_i, aug_j, idx_i, idx_j, out_i, out_j,
                        idx_v, rows_v, sem):
    wid = _wid()
    base = wid * s_per_tile
    pltpu.sync_copy(idx_i.at[pl.ds(base, s_per_tile)], idx_v)
    pltpu.async_copy(aug_i.at[idx_v], rows_v, sem).wait()
    pltpu.sync_copy(rows_v, out_i.at[pl.ds(base, s_per_tile)])
    pltpu.sync_copy(idx_j.at[pl.ds(base, s_per_tile)], idx_v)
    pltpu.async_copy(aug_j.at[idx_v], rows_v, sem).wait()
    pltpu.sync_copy(rows_v, out_j.at[pl.ds(base, s_per_tile)])


def _dense_body(s_i, s_j, a_ref, b_ref, o_ref):
    i = pl.program_id(0)
    a = a_ref[...]
    b = b_ref[...]
    az = a[:, :D] + 1e-6
    bz = b[:, :D]
    a2 = jnp.sum(az * az, axis=1, keepdims=True)
    b2 = jnp.sum(bz * bz, axis=1)[None, :]
    cross = lax.dot_general(az, bz, (((1,), (1,)), ((), ())),
                            preferred_element_type=jnp.float32)
    d2 = jnp.maximum(a2 + b2 - 2.0 * cross, 0.0)
    lam = a[:, D][:, None] + b[:, D][None, :] - jnp.sqrt(d2)
    n_pad = b.shape[0]
    rows = i * S_BLK + lax.broadcasted_iota(jnp.int32, (S_BLK, n_pad), 0)
    cols = lax.broadcasted_iota(jnp.int32, (S_BLK, n_pad), 1)
    val = jnp.sum(jnp.where((rows < s_i) & (cols < s_j), jnp.exp(lam), 0.0))

    @pl.when(i == 0)
    def _():
        o_ref[...] = jnp.zeros((1, 1), jnp.float32)

    o_ref[...] = o_ref[...] + val


def kernel(beta, gamma, latent_zi, latent_zj, sample_i_idx, sample_j_idx,
           sparse_i_sample, sparse_j_sample):
    n_i, d = latent_zi.shape
    n_j, _ = latent_zj.shape
    s_i = sample_i_idx.shape[0]
    s_j = sample_j_idx.shape[0]
    e = sparse_i_sample.shape[0]
    f32 = jnp.float32

    aug_i = jnp.concatenate(
        [latent_zi, beta[:, None], jnp.zeros((n_i, AUG - d - 1), f32)], axis=1)
    aug_j = jnp.concatenate(
        [latent_zj, gamma[:, None], jnp.zeros((n_j, AUG - d - 1), f32)], axis=1)

    e_per_tile = e // NW

    sc_params = pltpu.CompilerParams(use_tc_tiling_on_sc=False,
                                     needs_layout_passes=False)
    mesh = plsc.VectorSubcoreMesh(core_axis_name="c", subcore_axis_name="s",
                                  num_cores=NC, num_subcores=NS)
    link_fn = pl.kernel(
        functools.partial(_link_kernel_body, e_per_tile),
        out_type=jax.ShapeDtypeStruct((NW, LANES), f32),
        mesh=mesh,
        compiler_params=sc_params,
        scratch_types=[
            pltpu.VMEM((CHUNK,), jnp.int32),
            pltpu.VMEM((CHUNK,), jnp.int32),
            pltpu.VMEM((CHUNK, AUG), f32),
            pltpu.VMEM((CHUNK, AUG), f32),
            pltpu.VMEM((CHUNK,), jnp.int32),
            pltpu.VMEM((CHUNK,), jnp.int32),
            pltpu.VMEM((CHUNK, AUG), f32),
            pltpu.VMEM((CHUNK, AUG), f32),
            pltpu.VMEM((LANES,), f32),
            pltpu.SemaphoreType.DMA,
            pltpu.SemaphoreType.DMA,
            pltpu.SemaphoreType.DMA,
            pltpu.SemaphoreType.DMA,
        ],
    )
    s_pad = _ceil_to(max(s_i, s_j), NW * 8)
    s_per_tile = s_pad // NW
    sidx_i = jnp.pad(sample_i_idx.astype(jnp.int32), (0, s_pad - s_i))
    sidx_j = jnp.pad(sample_j_idx.astype(jnp.int32), (0, s_pad - s_j))
    sample_fn = pl.kernel(
        functools.partial(_sample_kernel_body, s_per_tile),
        out_type=(jax.ShapeDtypeStruct((s_pad, AUG), f32),
                  jax.ShapeDtypeStruct((s_pad, AUG), f32)),
        mesh=plsc.VectorSubcoreMesh(core_axis_name="c", subcore_axis_name="s",
                                    num_cores=NC, num_subcores=NS),
        compiler_params=sc_params,
        scratch_types=[
            pltpu.VMEM((s_per_tile,), jnp.int32),
            pltpu.VMEM((s_per_tile, AUG), f32),
            pltpu.SemaphoreType.DMA,
        ],
    )
    rows_i_s, rows_j_s = sample_fn(aug_i, aug_j, sidx_i, sidx_j)

    link_partials = link_fn(aug_i, aug_j,
                            sparse_i_sample.astype(jnp.int32),
                            sparse_j_sample.astype(jnp.int32))

    exp_sum = pl.pallas_call(
        functools.partial(_dense_body, s_i, s_j),
        grid=(s_pad // S_BLK,),
        in_specs=[
            pl.BlockSpec((S_BLK, AUG), lambda i: (i, 0)),
            pl.BlockSpec((s_pad, AUG), lambda i: (0, 0)),
        ],
        out_specs=pl.BlockSpec((1, 1), lambda i: (0, 0)),
        out_shape=jax.ShapeDtypeStruct((1, 1), f32),
    )(rows_i_s, rows_j_s)

    return jnp.sum(link_partials) - exp_sum[0, 0]


# speedup vs baseline: 1.6641x; 1.0086x over previous
import functools

import jax
import jax.numpy as jnp
from jax import lax
from jax.experimental import pallas as pl
from jax.experimental.pallas import tpu as pltpu
from jax.experimental.pallas import tpu_sc as plsc

NC = 2
NS = 16
NW = NC * NS
LANES = 16

D = 8
AUG = 16
SUB = 128
NSUB = 13
CHUNK = SUB * NSUB

S_BLK = 256


def _ceil_to(x, m):
    return (x + m - 1) // m * m


def _vsqrt(x):
    x = jnp.maximum(x, 1e-24)
    i = plsc.bitcast(x, jnp.int32)
    i = jnp.int32(0x5F3759DF) - lax.shift_right_logical(i, 1)
    r = plsc.bitcast(i, jnp.float32)
    r = r * (1.5 - 0.5 * x * r * r)
    r = r * (1.5 - 0.5 * x * r * r)
    r = r * (1.5 - 0.5 * x * r * r)
    return x * r


def _wid():
    return lax.axis_index("s") * NC + lax.axis_index("c")


def _link_kernel_body(e_per_tile, aug_i, aug_j, idx_i, idx_j, out,
                      idx_iv0, idx_jv0, rows_iv0, rows_jv0,
                      idx_iv1, idx_jv1, rows_iv1, rows_jv1,
                      acc_v, sem_i0, sem_j0, sem_i1, sem_j1):
    wid = _wid()
    base_e = wid * e_per_tile
    iota = lax.iota(jnp.int32, LANES)
    n_full = e_per_tile // CHUNK
    tail = e_per_tile - n_full * CHUNK
    tail_subs, off = [], 0
    while off < tail:
        sz = min(SUB, tail - off)
        tail_subs.append((off, sz))
        off += sz
    full_subs = [(k * SUB, SUB) for k in range(NSUB)]
    bufs = ((idx_iv0, idx_jv0, rows_iv0, rows_jv0, sem_i0, sem_j0),
            (idx_iv1, idx_jv1, rows_iv1, rows_jv1, sem_i1, sem_j1))

    def fire(eb, count, subs, buf):
        idx_iv, idx_jv, rows_iv, rows_jv, sem_i, sem_j = bufs[buf]
        pltpu.sync_copy(idx_i.at[pl.ds(eb, count)],
                        idx_iv.at[pl.ds(0, count)])
        pltpu.sync_copy(idx_j.at[pl.ds(eb, count)],
                        idx_jv.at[pl.ds(0, count)])
        cps = []
        for o, sz in subs:
            cps.append(pltpu.async_copy(
                aug_i.at[idx_iv.at[pl.ds(o, sz)]],
                rows_iv.at[pl.ds(o, sz)], sem_i))
            cps.append(pltpu.async_copy(
                aug_j.at[idx_jv.at[pl.ds(o, sz)]],
                rows_jv.at[pl.ds(o, sz)], sem_j))
        return cps

    def compute(n_groups, buf, acc):
        rows_iv, rows_jv = bufs[buf][2], bufs[buf][3]

        def group_body(g, acc):
            r = g * LANES + iota
            sq = []
            for d in range(D):
                col = jnp.full((LANES,), d, jnp.int32)
                a = plsc.load_gather(rows_iv, [r, col])
                b = plsc.load_gather(rows_jv, [r, col])
                diff = a - b + 1e-6
                sq.append(diff * diff)
            d2 = (((sq[0] + sq[1]) + (sq[2] + sq[3]))
                  + ((sq[4] + sq[5]) + (sq[6] + sq[7])))
            colb = jnp.full((LANES,), D, jnp.int32)
            beta_v = plsc.load_gather(rows_iv, [r, colb])
            gamma_v = plsc.load_gather(rows_jv, [r, colb])
            return acc + (beta_v + gamma_v - _vsqrt(d2))

        return lax.fori_loop(0, n_groups, group_body, acc)

    def drain(buf):
        idx_iv, idx_jv, rows_iv, rows_jv, sem_i, sem_j = bufs[buf]
        for o, sz in full_subs:
            pltpu.make_async_copy(
                aug_i.at[idx_iv.at[pl.ds(o, sz)]],
                rows_iv.at[pl.ds(o, sz)], sem_i).wait()
            pltpu.make_async_copy(
                aug_j.at[idx_jv.at[pl.ds(o, sz)]],
                rows_jv.at[pl.ds(o, sz)], sem_j).wait()

    n_pairs = n_full // 2
    fire(base_e, CHUNK, full_subs, 0)

    def outer(t, acc):
        for b in (0, 1):
            c = 2 * t + b

            @pl.when(c + 1 < n_full)
            def _():
                fire(base_e + (c + 1) * CHUNK, CHUNK, full_subs, 1 - b)

            drain(b)
            acc = compute(CHUNK // LANES, b, acc)
        return acc

    acc = lax.fori_loop(0, n_pairs, outer, jnp.zeros((LANES,), jnp.float32))
    if tail:
        for cp in fire(base_e + n_full * CHUNK, tail, tail_subs, 0):
            cp.wait()
        acc = compute(tail // LANES, 0, acc)
    acc_v[...] = acc
    pltpu.sync_copy(acc_v, out.at[wid])


def _sample_kernel_body(s_per_tile, aug_i, aug_j, idx_i, idx_j, out_i, out_j,
                        idx_v, rows_v, sem):
    wid = _wid()
    base = wid * s_per_tile
    pltpu.sync_copy(idx_i.at[pl.ds(base, s_per_tile)], idx_v)
    pltpu.async_copy(aug_i.at[idx_v], rows_v, sem).wait()
    pltpu.sync_copy(rows_v, out_i.at[pl.ds(base, s_per_tile)])
    pltpu.sync_copy(idx_j.at[pl.ds(base, s_per_tile)], idx_v)
    pltpu.async_copy(aug_j.at[idx_v], rows_v, sem).wait()
    pltpu.sync_copy(rows_v, out_j.at[pl.ds(base, s_per_tile)])


def _dense_body(s_i, s_j, a_ref, b_ref, o_ref):
    i = pl.program_id(0)
    a = a_ref[...]
    b = b_ref[...]
    az = a[:, :D] + 1e-6
    bz = b[:, :D]
    a2 = jnp.sum(az * az, axis=1, keepdims=True)
    b2 = jnp.sum(bz * bz, axis=1)[None, :]
    cross = lax.dot_general(az, bz, (((1,), (1,)), ((), ())),
                            preferred_element_type=jnp.float32)
    d2 = jnp.maximum(a2 + b2 - 2.0 * cross, 0.0)
    lam = a[:, D][:, None] + b[:, D][None, :] - jnp.sqrt(d2)
    n_pad = b.shape[0]
    rows = i * S_BLK + lax.broadcasted_iota(jnp.int32, (S_BLK, n_pad), 0)
    cols = lax.broadcasted_iota(jnp.int32, (S_BLK, n_pad), 1)
    val = jnp.sum(jnp.where((rows < s_i) & (cols < s_j), jnp.exp(lam), 0.0))

    @pl.when(i == 0)
    def _():
        o_ref[...] = jnp.zeros((1, 1), jnp.float32)

    o_ref[...] = o_ref[...] + val


def kernel(beta, gamma, latent_zi, latent_zj, sample_i_idx, sample_j_idx,
           sparse_i_sample, sparse_j_sample):
    n_i, d = latent_zi.shape
    n_j, _ = latent_zj.shape
    s_i = sample_i_idx.shape[0]
    s_j = sample_j_idx.shape[0]
    e = sparse_i_sample.shape[0]
    f32 = jnp.float32

    aug_i = jnp.concatenate(
        [latent_zi, beta[:, None], jnp.zeros((n_i, AUG - d - 1), f32)], axis=1)
    aug_j = jnp.concatenate(
        [latent_zj, gamma[:, None], jnp.zeros((n_j, AUG - d - 1), f32)], axis=1)

    e_per_tile = e // NW

    sc_params = pltpu.CompilerParams(use_tc_tiling_on_sc=False,
                                     needs_layout_passes=False)
    mesh = plsc.VectorSubcoreMesh(core_axis_name="c", subcore_axis_name="s",
                                  num_cores=NC, num_subcores=NS)
    link_fn = pl.kernel(
        functools.partial(_link_kernel_body, e_per_tile),
        out_type=jax.ShapeDtypeStruct((NW, LANES), f32),
        mesh=mesh,
        compiler_params=sc_params,
        scratch_types=[
            pltpu.VMEM((CHUNK,), jnp.int32),
            pltpu.VMEM((CHUNK,), jnp.int32),
            pltpu.VMEM((CHUNK, AUG), f32),
            pltpu.VMEM((CHUNK, AUG), f32),
            pltpu.VMEM((CHUNK,), jnp.int32),
            pltpu.VMEM((CHUNK,), jnp.int32),
            pltpu.VMEM((CHUNK, AUG), f32),
            pltpu.VMEM((CHUNK, AUG), f32),
            pltpu.VMEM((LANES,), f32),
            pltpu.SemaphoreType.DMA,
            pltpu.SemaphoreType.DMA,
            pltpu.SemaphoreType.DMA,
            pltpu.SemaphoreType.DMA,
        ],
    )
    s_pad = _ceil_to(max(s_i, s_j), NW * 8)
    s_per_tile = s_pad // NW
    sidx_i = jnp.pad(sample_i_idx.astype(jnp.int32), (0, s_pad - s_i))
    sidx_j = jnp.pad(sample_j_idx.astype(jnp.int32), (0, s_pad - s_j))
    sample_fn = pl.kernel(
        functools.partial(_sample_kernel_body, s_per_tile),
        out_type=(jax.ShapeDtypeStruct((s_pad, AUG), f32),
                  jax.ShapeDtypeStruct((s_pad, AUG), f32)),
        mesh=plsc.VectorSubcoreMesh(core_axis_name="c", subcore_axis_name="s",
                                    num_cores=NC, num_subcores=NS),
        compiler_params=sc_params,
        scratch_types=[
            pltpu.VMEM((s_per_tile,), jnp.int32),
            pltpu.VMEM((s_per_tile, AUG), f32),
            pltpu.SemaphoreType.DMA,
        ],
    )
    rows_i_s, rows_j_s = sample_fn(aug_i, aug_j, sidx_i, sidx_j)

    link_partials = link_fn(aug_i, aug_j, sparse_i_sample, sparse_j_sample)

    exp_sum = pl.pallas_call(
        functools.partial(_dense_body, s_i, s_j),
        grid=(s_pad // S_BLK,),
        in_specs=[
            pl.BlockSpec((S_BLK, AUG), lambda i: (i, 0)),
            pl.BlockSpec((s_pad, AUG), lambda i: (0, 0)),
        ],
        out_specs=pl.BlockSpec((1, 1), lambda i: (0, 0)),
        out_shape=jax.ShapeDtypeStruct((1, 1), f32),
    )(rows_i_s, rows_j_s)

    return jnp.sum(link_partials) - exp_sum[0, 0]
